# Initial kernel scaffold; baseline (speedup 1.0000x reference)
import jax
import jax.numpy as jnp
from jax.experimental import pallas as pl


def kernel(inputs, edge_index, W, attn_l, attn_r):
    raise NotImplementedError("write your pallas kernel here")


# speedup vs baseline: 44.1305x; 44.1305x over previous
import functools

import jax
import jax.numpy as jnp
from jax import lax
from jax.experimental import pallas as pl
from jax.experimental.pallas import tpu as pltpu
from jax.experimental.pallas import tpu_sc as plsc

N = 10000
E = 320000
IN_DIM = 128
H = 8
D = 16
ALPHA = 0.2

NC = 2
NS = 16
L = 16
NW = NC * NS

TW = IN_DIM + L
NPAD = 10240
PADROW = N

B = 128
NB = 80
IR = 8
EPT = NB * B
EPAD = EPT * NW

RPS = NPAD // NS
RPT = NPAD // NW
CB = 64


def _splat(vec, lane):
    idx = jnp.full((L, 1), lane, dtype=jnp.int32)
    return lax.gather(
        vec, idx,
        lax.GatherDimensionNumbers(
            offset_dims=(), collapsed_slice_dims=(0,), start_index_map=(0,)),
        (1,),
        mode=lax.GatherScatterMode.PROMISE_IN_BOUNDS)


def _project(xpad, c1, c2):
    BN = 1024

    def body(x_ref, c1_ref, c2_ref, t1_ref, t2_ref):
        xb = x_ref[...]
        t1_ref[...] = jnp.dot(xb, c1_ref[...], preferred_element_type=jnp.float32)
        t2_ref[...] = jnp.dot(xb, c2_ref[...], preferred_element_type=jnp.float32)

    return pl.pallas_call(
        body,
        grid=(NPAD // BN,),
        in_specs=[
            pl.BlockSpec((BN, IN_DIM), lambda i: (i, 0)),
            pl.BlockSpec((IN_DIM, TW), lambda i: (0, 0)),
            pl.BlockSpec((IN_DIM, L), lambda i: (0, 0)),
        ],
        out_specs=[
            pl.BlockSpec((BN, TW), lambda i: (i, 0)),
            pl.BlockSpec((BN, L), lambda i: (i, 0)),
        ],
        out_shape=[
            jax.ShapeDtypeStruct((NPAD, TW), jnp.float32),
            jax.ShapeDtypeStruct((NPAD, L), jnp.float32),
        ],
    )(xpad, c1, c2)


def _edge_mesh():
    return plsc.VectorSubcoreMesh(
        core_axis_name="c", subcore_axis_name="s", num_cores=NC, num_subcores=NS)


@functools.partial(
    pl.kernel,
    out_type=jax.ShapeDtypeStruct((NC, NPAD, TW), jnp.float32),
    mesh=_edge_mesh(),
    compiler_params=pltpu.CompilerParams(use_tc_tiling_on_sc=False),
    scratch_types=[
        pltpu.VMEM((IR, B), jnp.int32),
        pltpu.VMEM((IR, B), jnp.int32),
        pltpu.VMEM((B, TW), jnp.float32),
        pltpu.VMEM((B, L), jnp.float32),
        pltpu.VMEM_SHARED((NPAD, TW), jnp.float32),
        pltpu.SemaphoreType.DMA,
        pltpu.SemaphoreType.DMA,
    ],
)
def _edge_kernel(t1_hbm, t2_hbm, src_hbm, dst_hbm, zeros_hbm, out_hbm,
                 srcv, dstv, g1, g2, acc, sem1, sem2):
    cid = lax.axis_index("c")
    sid = lax.axis_index("s")
    wid = cid * NS + sid

    pltpu.sync_copy(zeros_hbm, acc.at[pl.ds(sid * RPS, RPS)])
    plsc.subcore_barrier()

    def outer(jo, carry):
        pltpu.sync_copy(src_hbm.at[wid, pl.ds(jo * IR, IR)], srcv)
        pltpu.sync_copy(dst_hbm.at[wid, pl.ds(jo * IR, IR)], dstv)

        def batch(j, c0):
            pltpu.async_copy(t1_hbm.at[srcv.at[j]], g1, sem1).wait()
            pltpu.async_copy(t2_hbm.at[dstv.at[j]], g2, sem2).wait()

            def edge(e, c):
                s = g1[e, pl.ds(IN_DIM, L)] + g2[e, :]
                av = jnp.minimum(jnp.exp(jnp.maximum(s, s * ALPHA)), 10.0)
                for h in range(H):
                    g1[e, pl.ds(h * D, D)] = g1[e, pl.ds(h * D, D)] * _splat(av, h)
                g1[e, pl.ds(IN_DIM, L)] = av
                return c

            lax.fori_loop(0, B, edge, 0, unroll=2)
            pltpu.sync_copy(g1, acc.at[dstv.at[j]], add=True)
            return c0

        lax.fori_loop(0, IR, batch, 0)
        return carry

    lax.fori_loop(0, NB // IR, outer, 0)

    plsc.subcore_barrier()
    pltpu.sync_copy(acc.at[pl.ds(sid * RPS, RPS)],
                    out_hbm.at[cid, pl.ds(sid * RPS, RPS)])


@functools.partial(
    pl.kernel,
    out_type=jax.ShapeDtypeStruct((NPAD, IN_DIM), jnp.float32),
    mesh=_edge_mesh(),
    compiler_params=pltpu.CompilerParams(use_tc_tiling_on_sc=False),
    scratch_types=[
        pltpu.VMEM((CB, TW), jnp.float32),
        pltpu.VMEM((CB, TW), jnp.float32),
        pltpu.VMEM((CB, IN_DIM), jnp.float32),
    ],
)
def _finalize_kernel(part_hbm, out_hbm, p0, p1, o):
    cid = lax.axis_index("c")
    sid = lax.axis_index("s")
    wid = cid * NS + sid
    base = wid * RPT

    def chunk(k, carry):
        row0 = base + k * CB
        pltpu.sync_copy(part_hbm.at[0, pl.ds(row0, CB)], p0)
        pltpu.sync_copy(part_hbm.at[1, pl.ds(row0, CB)], p1)

        def row(r, c):
            z = p0[r, pl.ds(IN_DIM, L)] + p1[r, pl.ds(IN_DIM, L)]
            for h in range(H):
                agg = p0[r, pl.ds(h * D, D)] + p1[r, pl.ds(h * D, D)]
                o[r, pl.ds(h * D, D)] = agg / _splat(z, h)
            return c

        lax.fori_loop(0, CB, row, 0, unroll=2)
        pltpu.sync_copy(o, out_hbm.at[pl.ds(row0, CB)])
        return carry

    lax.fori_loop(0, RPT // CB, chunk, 0)


def kernel(inputs, edge_index, W, attn_l, attn_r):
    al = attn_l.reshape(H, D)
    ar = attn_r.reshape(H, D)
    sel = jnp.eye(H, L, dtype=jnp.float32)
    Al = (al[:, :, None] * sel[:, None, :]).reshape(H * D, L)
    Ar = (ar[:, :, None] * sel[:, None, :]).reshape(H * D, L)
    WT = W.T
    c1 = jnp.concatenate([WT, WT @ Al], axis=1)
    c2 = WT @ Ar

    xpad = jnp.pad(inputs, ((0, NPAD - N), (0, 0)))
    t1, t2 = _project(xpad, c1, c2)

    ei = jnp.pad(edge_index, ((0, 0), (0, EPAD - E)), constant_values=PADROW)
    src = ei[0].reshape(NW, NB, B)
    dst = ei[1].reshape(NW, NB, B)
    zeros = jnp.zeros((RPS, TW), jnp.float32)

    partials = _edge_kernel(t1, t2, src, dst, zeros)
    out = _finalize_kernel(partials)
    return out[:N].reshape(N, H, D)


# speedup vs baseline: 77.9632x; 1.7667x over previous
import functools

import jax
import jax.numpy as jnp
from jax import lax
from jax.experimental import pallas as pl
from jax.experimental.pallas import tpu as pltpu
from jax.experimental.pallas import tpu_sc as plsc

N = 10000
E = 320000
IN_DIM = 128
H = 8
D = 16
ALPHA = 0.2

NC = 2
NS = 16
L = 16
NW = NC * NS

TW = IN_DIM + L
NPAD = 10240
PADROW = N

B = 112
NB = 90
IR = 6
EPT = NB * B
EPAD = EPT * NW

RPS = NPAD // NS
RPT = NPAD // NW
CB = 64


def _splat(vec, lane):
    idx = jnp.full((L, 1), lane, dtype=jnp.int32)
    return lax.gather(
        vec, idx,
        lax.GatherDimensionNumbers(
            offset_dims=(), collapsed_slice_dims=(0,), start_index_map=(0,)),
        (1,),
        mode=lax.GatherScatterMode.PROMISE_IN_BOUNDS)


def _project(xpad, c1, c2):
    BN = 1024

    def body(x_ref, c1_ref, c2_ref, t1_ref, t2_ref):
        xb = x_ref[...]
        t1_ref[...] = jnp.dot(xb, c1_ref[...], preferred_element_type=jnp.float32)
        t2_ref[...] = jnp.dot(xb, c2_ref[...], preferred_element_type=jnp.float32)

    return pl.pallas_call(
        body,
        grid=(NPAD // BN,),
        in_specs=[
            pl.BlockSpec((BN, IN_DIM), lambda i: (i, 0)),
            pl.BlockSpec((IN_DIM, TW), lambda i: (0, 0)),
            pl.BlockSpec((IN_DIM, L), lambda i: (0, 0)),
        ],
        out_specs=[
            pl.BlockSpec((BN, TW), lambda i: (i, 0)),
            pl.BlockSpec((BN, L), lambda i: (i, 0)),
        ],
        out_shape=[
            jax.ShapeDtypeStruct((NPAD, TW), jnp.float32),
            jax.ShapeDtypeStruct((NPAD, L), jnp.float32),
        ],
    )(xpad, c1, c2)


def _edge_mesh():
    return plsc.VectorSubcoreMesh(
        core_axis_name="c", subcore_axis_name="s", num_cores=NC, num_subcores=NS)


@functools.partial(
    pl.kernel,
    out_type=jax.ShapeDtypeStruct((NC, NPAD, TW), jnp.float32),
    mesh=_edge_mesh(),
    compiler_params=pltpu.CompilerParams(use_tc_tiling_on_sc=False),
    scratch_types=[
        pltpu.VMEM((IR, B), jnp.int32),
        pltpu.VMEM((IR, B), jnp.int32),
        pltpu.VMEM((2, B, TW), jnp.float32),
        pltpu.VMEM((2, B, L), jnp.float32),
        pltpu.VMEM_SHARED((NPAD, TW), jnp.float32),
        pltpu.SemaphoreType.DMA,
        pltpu.SemaphoreType.DMA,
        pltpu.SemaphoreType.DMA,
        pltpu.SemaphoreType.DMA,
    ],
)
def _edge_kernel(t1_hbm, t2_hbm, src_hbm, dst_hbm, zeros_hbm, out_hbm,
                 srcv, dstv, g1, g2, acc, s1a, s1b, s2a, s2b):
    cid = lax.axis_index("c")
    sid = lax.axis_index("s")
    wid = cid * NS + sid
    sem1 = [s1a, s1b]
    sem2 = [s2a, s2b]

    pltpu.sync_copy(zeros_hbm, acc.at[pl.ds(sid * RPS, RPS)])
    plsc.subcore_barrier()

    def outer(jo, carry):
        pltpu.sync_copy(src_hbm.at[wid, pl.ds(jo * IR, IR)], srcv)
        pltpu.sync_copy(dst_hbm.at[wid, pl.ds(jo * IR, IR)], dstv)

        h1 = [pltpu.async_copy(t1_hbm.at[srcv.at[0]], g1.at[0], sem1[0]), None]
        h2 = [pltpu.async_copy(t2_hbm.at[dstv.at[0]], g2.at[0], sem2[0]), None]
        for b in range(IR):
            p = b % 2
            q = 1 - p
            if b + 1 < IR:
                h1[q] = pltpu.async_copy(t1_hbm.at[srcv.at[b + 1]], g1.at[q], sem1[q])
                h2[q] = pltpu.async_copy(t2_hbm.at[dstv.at[b + 1]], g2.at[q], sem2[q])
            h1[p].wait()
            h2[p].wait()

            def edge(e, c):
                s = g1[p, e, pl.ds(IN_DIM, L)] + g2[p, e, :]
                av = jnp.minimum(jnp.exp(jnp.maximum(s, s * ALPHA)), 10.0)
                for h in range(H):
                    g1[p, e, pl.ds(h * D, D)] = g1[p, e, pl.ds(h * D, D)] * _splat(av, h)
                g1[p, e, pl.ds(IN_DIM, L)] = av
                return c

            lax.fori_loop(0, B, edge, 0, unroll=2)
            pltpu.sync_copy(g1.at[p], acc.at[dstv.at[b]], add=True)
        return carry

    lax.fori_loop(0, NB // IR, outer, 0)

    plsc.subcore_barrier()
    pltpu.sync_copy(acc.at[pl.ds(sid * RPS, RPS)],
                    out_hbm.at[cid, pl.ds(sid * RPS, RPS)])


@functools.partial(
    pl.kernel,
    out_type=jax.ShapeDtypeStruct((NPAD, IN_DIM), jnp.float32),
    mesh=_edge_mesh(),
    compiler_params=pltpu.CompilerParams(use_tc_tiling_on_sc=False),
    scratch_types=[
        pltpu.VMEM((CB, TW), jnp.float32),
        pltpu.VMEM((CB, TW), jnp.float32),
        pltpu.VMEM((CB, IN_DIM), jnp.float32),
    ],
)
def _finalize_kernel(part_hbm, out_hbm, p0, p1, o):
    cid = lax.axis_index("c")
    sid = lax.axis_index("s")
    wid = cid * NS + sid
    base = wid * RPT

    def chunk(k, carry):
        row0 = base + k * CB
        pltpu.sync_copy(part_hbm.at[0, pl.ds(row0, CB)], p0)
        pltpu.sync_copy(part_hbm.at[1, pl.ds(row0, CB)], p1)

        def row(r, c):
            z = p0[r, pl.ds(IN_DIM, L)] + p1[r, pl.ds(IN_DIM, L)]
            for h in range(H):
                agg = p0[r, pl.ds(h * D, D)] + p1[r, pl.ds(h * D, D)]
                o[r, pl.ds(h * D, D)] = agg / _splat(z, h)
            return c

        lax.fori_loop(0, CB, row, 0, unroll=2)
        pltpu.sync_copy(o, out_hbm.at[pl.ds(row0, CB)])
        return carry

    lax.fori_loop(0, RPT // CB, chunk, 0)


def kernel(inputs, edge_index, W, attn_l, attn_r):
    al = attn_l.reshape(H, D)
    ar = attn_r.reshape(H, D)
    sel = jnp.eye(H, L, dtype=jnp.float32)
    Al = (al[:, :, None] * sel[:, None, :]).reshape(H * D, L)
    Ar = (ar[:, :, None] * sel[:, None, :]).reshape(H * D, L)
    WT = W.T
    c1 = jnp.concatenate([WT, WT @ Al], axis=1)
    c2 = WT @ Ar

    xpad = jnp.pad(inputs, ((0, NPAD - N), (0, 0)))
    t1, t2 = _project(xpad, c1, c2)

    ei = jnp.pad(edge_index, ((0, 0), (0, EPAD - E)), constant_values=PADROW)
    src = ei[0].reshape(NW, NB, B)
    dst = ei[1].reshape(NW, NB, B)
    zeros = jnp.zeros((RPS, TW), jnp.float32)

    partials = _edge_kernel(t1, t2, src, dst, zeros)
    out = _finalize_kernel(partials)
    return out[:N].reshape(N, H, D)


# speedup vs baseline: 78.8692x; 1.0116x over previous
import functools

import jax
import jax.numpy as jnp
from jax import lax
from jax.experimental import pallas as pl
from jax.experimental.pallas import tpu as pltpu
from jax.experimental.pallas import tpu_sc as plsc

N = 10000
E = 320000
IN_DIM = 128
H = 8
D = 16
ALPHA = 0.2

NC = 2
NS = 16
L = 16
NW = NC * NS

TW = IN_DIM + L
NPAD = 10240
PADROW = N

B = 112
NB = 90
IR = 6
EPT = NB * B
EPAD = EPT * NW

RPS = NPAD // NS
RPT = NPAD // NW
CB = 64


def _splat(vec, lane):
    idx = jnp.full((L, 1), lane, dtype=jnp.int32)
    return lax.gather(
        vec, idx,
        lax.GatherDimensionNumbers(
            offset_dims=(), collapsed_slice_dims=(0,), start_index_map=(0,)),
        (1,),
        mode=lax.GatherScatterMode.PROMISE_IN_BOUNDS)


def _project(xpad, c1, c2):
    BN = 1024

    def body(x_ref, c1_ref, c2_ref, t1_ref, t2_ref):
        xb = x_ref[...]
        t1_ref[...] = jnp.dot(xb, c1_ref[...], preferred_element_type=jnp.float32)
        t2_ref[...] = jnp.dot(xb, c2_ref[...], preferred_element_type=jnp.float32)

    return pl.pallas_call(
        body,
        grid=(NPAD // BN,),
        in_specs=[
            pl.BlockSpec((BN, IN_DIM), lambda i: (i, 0)),
            pl.BlockSpec((IN_DIM, TW), lambda i: (0, 0)),
            pl.BlockSpec((IN_DIM, L), lambda i: (0, 0)),
        ],
        out_specs=[
            pl.BlockSpec((BN, TW), lambda i: (i, 0)),
            pl.BlockSpec((BN, L), lambda i: (i, 0)),
        ],
        out_shape=[
            jax.ShapeDtypeStruct((NPAD, TW), jnp.float32),
            jax.ShapeDtypeStruct((NPAD, L), jnp.float32),
        ],
    )(xpad, c1, c2)


def _edge_mesh():
    return plsc.VectorSubcoreMesh(
        core_axis_name="c", subcore_axis_name="s", num_cores=NC, num_subcores=NS)


@functools.partial(
    pl.kernel,
    out_type=jax.ShapeDtypeStruct((NC, NPAD, TW), jnp.float32),
    mesh=_edge_mesh(),
    compiler_params=pltpu.CompilerParams(use_tc_tiling_on_sc=False),
    scratch_types=[
        pltpu.VMEM((IR, B), jnp.int32),
        pltpu.VMEM((IR, B), jnp.int32),
        pltpu.VMEM((2, B, TW), jnp.float32),
        pltpu.VMEM((2, B, L), jnp.float32),
        pltpu.VMEM_SHARED((NPAD, TW), jnp.float32),
        pltpu.SemaphoreType.DMA,
        pltpu.SemaphoreType.DMA,
        pltpu.SemaphoreType.DMA,
        pltpu.SemaphoreType.DMA,
        pltpu.SemaphoreType.DMA,
        pltpu.SemaphoreType.DMA,
    ],
)
def _edge_kernel(t1_hbm, t2_hbm, src_hbm, dst_hbm, zeros_hbm, out_hbm,
                 srcv, dstv, g1, g2, acc, s1a, s1b, s2a, s2b, s3a, s3b):
    cid = lax.axis_index("c")
    sid = lax.axis_index("s")
    wid = cid * NS + sid
    sem1 = [s1a, s1b]
    sem2 = [s2a, s2b]
    sem3 = [s3a, s3b]

    pltpu.sync_copy(zeros_hbm, acc.at[pl.ds(sid * RPS, RPS)])
    plsc.subcore_barrier()

    def outer(jo, carry):
        pltpu.sync_copy(src_hbm.at[wid, pl.ds(jo * IR, IR)], srcv)
        pltpu.sync_copy(dst_hbm.at[wid, pl.ds(jo * IR, IR)], dstv)

        h1 = [pltpu.async_copy(t1_hbm.at[srcv.at[0]], g1.at[0], sem1[0]), None]
        h2 = [pltpu.async_copy(t2_hbm.at[dstv.at[0]], g2.at[0], sem2[0]), None]
        h3 = [None, None]
        for b in range(IR):
            p = b % 2
            q = 1 - p
            if b + 1 < IR:
                if h3[q] is not None:
                    h3[q].wait()
                h1[q] = pltpu.async_copy(t1_hbm.at[srcv.at[b + 1]], g1.at[q], sem1[q])
                h2[q] = pltpu.async_copy(t2_hbm.at[dstv.at[b + 1]], g2.at[q], sem2[q])
            h1[p].wait()
            h2[p].wait()

            def edge(e, c):
                s = g1[p, e, pl.ds(IN_DIM, L)] + g2[p, e, :]
                av = jnp.minimum(jnp.exp(jnp.maximum(s, s * ALPHA)), 10.0)
                for h in range(H):
                    g1[p, e, pl.ds(h * D, D)] = g1[p, e, pl.ds(h * D, D)] * _splat(av, h)
                g1[p, e, pl.ds(IN_DIM, L)] = av
                return c

            lax.fori_loop(0, B, edge, 0, unroll=2)
            h3[p] = pltpu.async_copy(g1.at[p], acc.at[dstv.at[b]], sem3[p], add=True)
        h3[0].wait()
        h3[1].wait()
        return carry

    lax.fori_loop(0, NB // IR, outer, 0)

    plsc.subcore_barrier()
    pltpu.sync_copy(acc.at[pl.ds(sid * RPS, RPS)],
                    out_hbm.at[cid, pl.ds(sid * RPS, RPS)])


@functools.partial(
    pl.kernel,
    out_type=jax.ShapeDtypeStruct((NPAD, IN_DIM), jnp.float32),
    mesh=_edge_mesh(),
    compiler_params=pltpu.CompilerParams(use_tc_tiling_on_sc=False),
    scratch_types=[
        pltpu.VMEM((CB, TW), jnp.float32),
        pltpu.VMEM((CB, TW), jnp.float32),
        pltpu.VMEM((CB, IN_DIM), jnp.float32),
    ],
)
def _finalize_kernel(part_hbm, out_hbm, p0, p1, o):
    cid = lax.axis_index("c")
    sid = lax.axis_index("s")
    wid = cid * NS + sid
    base = wid * RPT

    def chunk(k, carry):
        row0 = base + k * CB
        pltpu.sync_copy(part_hbm.at[0, pl.ds(row0, CB)], p0)
        pltpu.sync_copy(part_hbm.at[1, pl.ds(row0, CB)], p1)

        def row(r, c):
            z = p0[r, pl.ds(IN_DIM, L)] + p1[r, pl.ds(IN_DIM, L)]
            for h in range(H):
                agg = p0[r, pl.ds(h * D, D)] + p1[r, pl.ds(h * D, D)]
                o[r, pl.ds(h * D, D)] = agg / _splat(z, h)
            return c

        lax.fori_loop(0, CB, row, 0, unroll=2)
        pltpu.sync_copy(o, out_hbm.at[pl.ds(row0, CB)])
        return carry

    lax.fori_loop(0, RPT // CB, chunk, 0)


def kernel(inputs, edge_index, W, attn_l, attn_r):
    al = attn_l.reshape(H, D)
    ar = attn_r.reshape(H, D)
    sel = jnp.eye(H, L, dtype=jnp.float32)
    Al = (al[:, :, None] * sel[:, None, :]).reshape(H * D, L)
    Ar = (ar[:, :, None] * sel[:, None, :]).reshape(H * D, L)
    WT = W.T
    c1 = jnp.concatenate([WT, WT @ Al], axis=1)
    c2 = WT @ Ar

    xpad = jnp.pad(inputs, ((0, NPAD - N), (0, 0)))
    t1, t2 = _project(xpad, c1, c2)

    ei = jnp.pad(edge_index, ((0, 0), (0, EPAD - E)), constant_values=PADROW)
    src = ei[0].reshape(NW, NB, B)
    dst = ei[1].reshape(NW, NB, B)
    zeros = jnp.zeros((RPS, TW), jnp.float32)

    partials = _edge_kernel(t1, t2, src, dst, zeros)
    out = _finalize_kernel(partials)
    return out[:N].reshape(N, H, D)


# speedup vs baseline: 97.8925x; 1.2412x over previous
import functools

import jax
import jax.numpy as jnp
from jax import lax
from jax.experimental import pallas as pl
from jax.experimental.pallas import tpu as pltpu
from jax.experimental.pallas import tpu_sc as plsc

N = 10000
E = 320000
IN_DIM = 128
H = 8
D = 16
ALPHA = 0.2

NC = 2
NS = 16
L = 16
NW = NC * NS

TW = IN_DIM + L
NPAD = 10240
PADROW = N

B = 112
NB = 90
IR = 6
EPT = NB * B
EPAD = EPT * NW

RPS = NPAD // NS
RPT = NPAD // NW
CB = 64


def _splat(vec, lane):
    idx = jnp.full((L, 1), lane, dtype=jnp.int32)
    return lax.gather(
        vec, idx,
        lax.GatherDimensionNumbers(
            offset_dims=(), collapsed_slice_dims=(0,), start_index_map=(0,)),
        (1,),
        mode=lax.GatherScatterMode.PROMISE_IN_BOUNDS)


def _project(xpad, c1, c2):
    BN = 1024

    def body(x_ref, c1_ref, c2_ref, t1_ref, t2_ref):
        xb = x_ref[...]
        t1_ref[...] = jnp.dot(xb, c1_ref[...], preferred_element_type=jnp.float32)
        t2_ref[...] = jnp.dot(xb, c2_ref[...], preferred_element_type=jnp.float32)

    return pl.pallas_call(
        body,
        grid=(NPAD // BN,),
        in_specs=[
            pl.BlockSpec((BN, IN_DIM), lambda i: (i, 0)),
            pl.BlockSpec((IN_DIM, TW), lambda i: (0, 0)),
            pl.BlockSpec((IN_DIM, L), lambda i: (0, 0)),
        ],
        out_specs=[
            pl.BlockSpec((BN, TW), lambda i: (i, 0)),
            pl.BlockSpec((BN, L), lambda i: (i, 0)),
        ],
        out_shape=[
            jax.ShapeDtypeStruct((NPAD, TW), jnp.float32),
            jax.ShapeDtypeStruct((NPAD, L), jnp.float32),
        ],
    )(xpad, c1, c2)


def _edge_mesh():
    return plsc.VectorSubcoreMesh(
        core_axis_name="c", subcore_axis_name="s", num_cores=NC, num_subcores=NS)


@functools.partial(
    pl.kernel,
    out_type=jax.ShapeDtypeStruct((NC, NPAD, TW), jnp.float32),
    mesh=_edge_mesh(),
    compiler_params=pltpu.CompilerParams(use_tc_tiling_on_sc=False),
    scratch_types=[
        pltpu.VMEM((IR, B), jnp.int32),
        pltpu.VMEM((IR, B), jnp.int32),
        pltpu.VMEM((2, B, TW), jnp.float32),
        pltpu.VMEM((2, B, L), jnp.float32),
        pltpu.VMEM_SHARED((NPAD, TW), jnp.float32),
        pltpu.SemaphoreType.DMA,
        pltpu.SemaphoreType.DMA,
        pltpu.SemaphoreType.DMA,
        pltpu.SemaphoreType.DMA,
        pltpu.SemaphoreType.DMA,
        pltpu.SemaphoreType.DMA,
    ],
)
def _edge_kernel(t1_hbm, t2_hbm, src_hbm, dst_hbm, zeros_hbm, out_hbm,
                 srcv, dstv, g1, g2, acc, s1a, s1b, s2a, s2b, s3a, s3b):
    cid = lax.axis_index("c")
    sid = lax.axis_index("s")
    wid = cid * NS + sid
    sem1 = [s1a, s1b]
    sem2 = [s2a, s2b]
    sem3 = [s3a, s3b]

    pltpu.sync_copy(zeros_hbm, acc.at[pl.ds(sid * RPS, RPS)])
    plsc.subcore_barrier()

    def outer(jo, carry):
        pltpu.sync_copy(src_hbm.at[wid, pl.ds(jo * IR, IR)], srcv)
        pltpu.sync_copy(dst_hbm.at[wid, pl.ds(jo * IR, IR)], dstv)

        h1 = [pltpu.async_copy(t1_hbm.at[srcv.at[0]], g1.at[0], sem1[0]), None]
        h2 = [pltpu.async_copy(t2_hbm.at[dstv.at[0]], g2.at[0], sem2[0]), None]
        h3 = [None, None]
        for b in range(IR):
            p = b % 2
            q = 1 - p
            if b + 1 < IR:
                if h3[q] is not None:
                    h3[q].wait()
                h1[q] = pltpu.async_copy(t1_hbm.at[srcv.at[b + 1]], g1.at[q], sem1[q])
                h2[q] = pltpu.async_copy(t2_hbm.at[dstv.at[b + 1]], g2.at[q], sem2[q])
            h1[p].wait()
            h2[p].wait()

            def edge(e):
                s = g1[p, e, pl.ds(IN_DIM, L)] + g2[p, e, :]
                av = jnp.minimum(jnp.exp(jnp.maximum(s, s * ALPHA)), 10.0)
                for h in range(H):
                    g1[p, e, pl.ds(h * D, D)] = g1[p, e, pl.ds(h * D, D)] * _splat(av, h)
                g1[p, e, pl.ds(IN_DIM, L)] = av

            plsc.parallel_loop(0, B, unroll=4)(edge)
            h3[p] = pltpu.async_copy(g1.at[p], acc.at[dstv.at[b]], sem3[p], add=True)
        h3[0].wait()
        h3[1].wait()
        return carry

    lax.fori_loop(0, NB // IR, outer, 0)

    plsc.subcore_barrier()
    pltpu.sync_copy(acc.at[pl.ds(sid * RPS, RPS)],
                    out_hbm.at[cid, pl.ds(sid * RPS, RPS)])


@functools.partial(
    pl.kernel,
    out_type=jax.ShapeDtypeStruct((NPAD, IN_DIM), jnp.float32),
    mesh=_edge_mesh(),
    compiler_params=pltpu.CompilerParams(use_tc_tiling_on_sc=False),
    scratch_types=[
        pltpu.VMEM((CB, TW), jnp.float32),
        pltpu.VMEM((CB, TW), jnp.float32),
        pltpu.VMEM((CB, IN_DIM), jnp.float32),
    ],
)
def _finalize_kernel(part_hbm, out_hbm, p0, p1, o):
    cid = lax.axis_index("c")
    sid = lax.axis_index("s")
    wid = cid * NS + sid
    base = wid * RPT

    def chunk(k, carry):
        row0 = base + k * CB
        pltpu.sync_copy(part_hbm.at[0, pl.ds(row0, CB)], p0)
        pltpu.sync_copy(part_hbm.at[1, pl.ds(row0, CB)], p1)

        def row(r, c):
            z = p0[r, pl.ds(IN_DIM, L)] + p1[r, pl.ds(IN_DIM, L)]
            for h in range(H):
                agg = p0[r, pl.ds(h * D, D)] + p1[r, pl.ds(h * D, D)]
                o[r, pl.ds(h * D, D)] = agg / _splat(z, h)
            return c

        lax.fori_loop(0, CB, row, 0, unroll=2)
        pltpu.sync_copy(o, out_hbm.at[pl.ds(row0, CB)])
        return carry

    lax.fori_loop(0, RPT // CB, chunk, 0)


def kernel(inputs, edge_index, W, attn_l, attn_r):
    al = attn_l.reshape(H, D)
    ar = attn_r.reshape(H, D)
    sel = jnp.eye(H, L, dtype=jnp.float32)
    Al = (al[:, :, None] * sel[:, None, :]).reshape(H * D, L)
    Ar = (ar[:, :, None] * sel[:, None, :]).reshape(H * D, L)
    WT = W.T
    c1 = jnp.concatenate([WT, WT @ Al], axis=1)
    c2 = WT @ Ar

    xpad = jnp.pad(inputs, ((0, NPAD - N), (0, 0)))
    t1, t2 = _project(xpad, c1, c2)

    ei = jnp.pad(edge_index, ((0, 0), (0, EPAD - E)), constant_values=PADROW)
    src = ei[0].reshape(NW, NB, B)
    dst = ei[1].reshape(NW, NB, B)
    zeros = jnp.zeros((RPS, TW), jnp.float32)

    partials = _edge_kernel(t1, t2, src, dst, zeros)
    out = _finalize_kernel(partials)
    return out[:N].reshape(N, H, D)


# speedup vs baseline: 101.4311x; 1.0361x over previous
import functools

import jax
import jax.numpy as jnp
from jax import lax
from jax.experimental import pallas as pl
from jax.experimental.pallas import tpu as pltpu
from jax.experimental.pallas import tpu_sc as plsc

N = 10000
E = 320000
IN_DIM = 128
H = 8
D = 16
ALPHA = 0.2

NC = 2
NS = 16
L = 16
NW = NC * NS

TW = IN_DIM + L
NPAD = 10240
PADROW = N

B = 112
NB = 90
IR = 6
EPT = NB * B
EPAD = EPT * NW

RPS = NPAD // NS
RPT = NPAD // NW
CB = 64


def _splat(vec, lane):
    idx = jnp.full((L, 1), lane, dtype=jnp.int32)
    return lax.gather(
        vec, idx,
        lax.GatherDimensionNumbers(
            offset_dims=(), collapsed_slice_dims=(0,), start_index_map=(0,)),
        (1,),
        mode=lax.GatherScatterMode.PROMISE_IN_BOUNDS)


def _project(xpad, c1, c2):
    BN = 1024

    def body(x_ref, c1_ref, c2_ref, t1_ref, t2_ref):
        xb = x_ref[...]
        t1_ref[...] = jnp.dot(xb, c1_ref[...], preferred_element_type=jnp.float32)
        t2_ref[...] = jnp.dot(xb, c2_ref[...], preferred_element_type=jnp.float32)

    return pl.pallas_call(
        body,
        grid=(NPAD // BN,),
        in_specs=[
            pl.BlockSpec((BN, IN_DIM), lambda i: (i, 0)),
            pl.BlockSpec((IN_DIM, TW), lambda i: (0, 0)),
            pl.BlockSpec((IN_DIM, L), lambda i: (0, 0)),
        ],
        out_specs=[
            pl.BlockSpec((BN, TW), lambda i: (i, 0)),
            pl.BlockSpec((BN, L), lambda i: (i, 0)),
        ],
        out_shape=[
            jax.ShapeDtypeStruct((NPAD, TW), jnp.float32),
            jax.ShapeDtypeStruct((NPAD, L), jnp.float32),
        ],
    )(xpad, c1, c2)


def _edge_mesh():
    return plsc.VectorSubcoreMesh(
        core_axis_name="c", subcore_axis_name="s", num_cores=NC, num_subcores=NS)


@functools.partial(
    pl.kernel,
    out_type=jax.ShapeDtypeStruct((NC, NPAD, TW), jnp.float32),
    mesh=_edge_mesh(),
    compiler_params=pltpu.CompilerParams(use_tc_tiling_on_sc=False),
    scratch_types=[
        pltpu.VMEM((IR, B), jnp.int32),
        pltpu.VMEM((IR, B), jnp.int32),
        pltpu.VMEM((2, B, TW), jnp.float32),
        pltpu.VMEM((2, B, L), jnp.float32),
        pltpu.VMEM_SHARED((NPAD, TW), jnp.float32),
        pltpu.SemaphoreType.DMA,
        pltpu.SemaphoreType.DMA,
        pltpu.SemaphoreType.DMA,
        pltpu.SemaphoreType.DMA,
        pltpu.SemaphoreType.DMA,
        pltpu.SemaphoreType.DMA,
    ],
)
def _edge_kernel(t1_hbm, t2_hbm, src_hbm, dst_hbm, zeros_hbm, out_hbm,
                 srcv, dstv, g1, g2, acc, s1a, s1b, s2a, s2b, s3a, s3b):
    cid = lax.axis_index("c")
    sid = lax.axis_index("s")
    wid = cid * NS + sid
    sem1 = [s1a, s1b]
    sem2 = [s2a, s2b]
    sem3 = [s3a, s3b]

    pltpu.sync_copy(zeros_hbm, acc.at[pl.ds(sid * RPS, RPS)])
    plsc.subcore_barrier()

    def outer(jo, carry):
        pltpu.sync_copy(src_hbm.at[wid, pl.ds(jo * IR, IR)], srcv)
        pltpu.sync_copy(dst_hbm.at[wid, pl.ds(jo * IR, IR)], dstv)

        h1 = [pltpu.async_copy(t1_hbm.at[srcv.at[0]], g1.at[0], sem1[0]), None]
        h2 = [pltpu.async_copy(t2_hbm.at[dstv.at[0]], g2.at[0], sem2[0]), None]
        h3 = [None, None]
        for b in range(IR):
            p = b % 2
            q = 1 - p
            if b + 1 < IR:
                if h3[q] is not None:
                    h3[q].wait()
                h1[q] = pltpu.async_copy(t1_hbm.at[srcv.at[b + 1]], g1.at[q], sem1[q])
                h2[q] = pltpu.async_copy(t2_hbm.at[dstv.at[b + 1]], g2.at[q], sem2[q])
            h1[p].wait()
            h2[p].wait()

            def edge(e):
                s = g1[p, e, pl.ds(IN_DIM, L)] + g2[p, e, :]
                av = jnp.minimum(jnp.exp(jnp.maximum(s, s * ALPHA)), 10.0)
                for h in range(H):
                    g1[p, e, pl.ds(h * D, D)] = g1[p, e, pl.ds(h * D, D)] * _splat(av, h)
                g1[p, e, pl.ds(IN_DIM, L)] = av

            plsc.parallel_loop(0, B, unroll=4)(edge)
            h3[p] = pltpu.async_copy(g1.at[p], acc.at[dstv.at[b]], sem3[p], add=True)
        h3[0].wait()
        h3[1].wait()
        return carry

    lax.fori_loop(0, NB // IR, outer, 0)

    plsc.subcore_barrier()
    pltpu.sync_copy(acc.at[pl.ds(sid * RPS, RPS)],
                    out_hbm.at[cid, pl.ds(sid * RPS, RPS)])


@functools.partial(
    pl.kernel,
    out_type=jax.ShapeDtypeStruct((NPAD, IN_DIM), jnp.float32),
    mesh=_edge_mesh(),
    compiler_params=pltpu.CompilerParams(use_tc_tiling_on_sc=False),
    scratch_types=[
        pltpu.VMEM((CB, TW), jnp.float32),
        pltpu.VMEM((CB, TW), jnp.float32),
        pltpu.VMEM((CB, IN_DIM), jnp.float32),
    ],
)
def _finalize_kernel(part_hbm, out_hbm, p0, p1, o):
    cid = lax.axis_index("c")
    sid = lax.axis_index("s")
    wid = cid * NS + sid
    base = wid * RPT

    def chunk(k, carry):
        row0 = base + k * CB
        pltpu.sync_copy(part_hbm.at[0, pl.ds(row0, CB)], p0)
        pltpu.sync_copy(part_hbm.at[1, pl.ds(row0, CB)], p1)

        def row(r):
            z = p0[r, pl.ds(IN_DIM, L)] + p1[r, pl.ds(IN_DIM, L)]
            for h in range(H):
                agg = p0[r, pl.ds(h * D, D)] + p1[r, pl.ds(h * D, D)]
                o[r, pl.ds(h * D, D)] = agg / _splat(z, h)

        plsc.parallel_loop(0, CB, unroll=4)(row)
        pltpu.sync_copy(o, out_hbm.at[pl.ds(row0, CB)])
        return carry

    lax.fori_loop(0, RPT // CB, chunk, 0)


def kernel(inputs, edge_index, W, attn_l, attn_r):
    al = attn_l.reshape(H, D)
    ar = attn_r.reshape(H, D)
    sel = jnp.eye(H, L, dtype=jnp.float32)
    Al = (al[:, :, None] * sel[:, None, :]).reshape(H * D, L)
    Ar = (ar[:, :, None] * sel[:, None, :]).reshape(H * D, L)
    WT = W.T
    c1 = jnp.concatenate([WT, WT @ Al], axis=1)
    c2 = WT @ Ar

    xpad = jnp.pad(inputs, ((0, NPAD - N), (0, 0)))
    t1, t2 = _project(xpad, c1, c2)

    ei = jnp.pad(edge_index, ((0, 0), (0, EPAD - E)), constant_values=PADROW)
    src = ei[0].reshape(NW, NB, B)
    dst = ei[1].reshape(NW, NB, B)
    zeros = jnp.zeros((RPS, TW), jnp.float32)

    partials = _edge_kernel(t1, t2, src, dst, zeros)
    out = _finalize_kernel(partials)
    return out[:N].reshape(N, H, D)
